# Initial kernel scaffold; baseline (speedup 1.0000x reference)
import jax
import jax.numpy as jnp
from jax.experimental import pallas as pl


def kernel(y_pred, targets):
    raise NotImplementedError("write your pallas kernel here")


# speedup vs baseline: 49.5940x; 49.5940x over previous
import jax
import jax.numpy as jnp
from jax import lax
from jax.experimental import pallas as pl
from jax.experimental.pallas import tpu as pltpu
from jax.experimental.pallas import tpu_sc as plsc

NBINS = 1024
NCORES = 2
NSUB = 16
LANES = 16
NCLASS = 21
CPC = 11
HPP = 512 * 512
NIMG = 8
NPIX = NIMG * HPP
PIX_PER_TILE = NPIX // NSUB
CS = 4096
NCHUNK = PIX_PER_TILE // CS
VPC = CS // LANES
HSZ = CPC * 2 * NBINS
CB = 2 * NBINS


def _body(yp_hbm, lab_hbm, out_hbm, hist, labv, pv, comb, tmp, outv, shared):
    core = lax.axis_index("c")
    sub = lax.axis_index("s")

    zeros16 = jnp.zeros((LANES,), jnp.float32)
    ones16 = jnp.ones((LANES,), jnp.float32)
    iota16 = lax.iota(jnp.int32, LANES)

    def zero_hist(i, _):
        hist[pl.ds(i * LANES, LANES)] = zeros16
        return 0

    lax.fori_loop(0, HSZ // LANES, zero_hist, 0)

    pixbase = sub * PIX_PER_TILE
    img = pixbase // HPP
    poff = pixbase - img * HPP

    def chunk_body(j, _):
        pltpu.sync_copy(lab_hbm.at[pl.ds(pixbase + j * CS, CS)], labv)
        for ci in range(CPC):
            cdyn = core * CPC + ci

            @pl.when(cdyn < NCLASS)
            def _process():
                yp_off = (img * NCLASS + cdyn) * HPP + poff + j * CS
                pltpu.sync_copy(yp_hbm.at[pl.ds(yp_off, CS)], pv)
                base_all = ci * CB

                def vbody(v, _):
                    p = pv[pl.ds(v * LANES, LANES)]
                    labs = labv[pl.ds(v * LANES, LANES)]
                    k = jnp.minimum((p * NBINS).astype(jnp.int32), NBINS - 1)
                    idx = k + base_all
                    plsc.addupdate_scatter(hist, [idx], ones16)
                    plsc.addupdate_scatter(hist, [idx + NBINS], ones16,
                                           mask=labs == cdyn)
                    return 0

                lax.fori_loop(0, VPC, vbody, 0)

        return 0

    lax.fori_loop(0, NCHUNK, chunk_body, 0)

    pltpu.sync_copy(hist, shared.at[pl.ds(sub * HSZ, HSZ)])
    plsc.subcore_barrier()

    cmine = core * CPC + sub

    @pl.when(jnp.logical_and(sub < CPC, cmine < NCLASS))
    def _scan():
        def zero_comb(i, _):
            comb[pl.ds(i * LANES, LANES)] = zeros16
            return 0

        lax.fori_loop(0, CB // LANES, zero_comb, 0)

        def tile_sum(t, _):
            pltpu.sync_copy(shared.at[pl.ds(t * HSZ + sub * CB, CB)], tmp)

            def add1(i, _):
                comb[pl.ds(i * LANES, LANES)] += tmp[pl.ds(i * LANES, LANES)]
                return 0

            lax.fori_loop(0, CB // LANES, add1, 0)
            return 0

        lax.fori_loop(0, NSUB, tile_sum, 0)

        def gsum(i, acc):
            return acc + jnp.sum(comb[pl.ds(NBINS + i * LANES, LANES)])

        G = lax.fori_loop(0, NBINS // LANES, gsum, jnp.float32(0.0))

        lanes_f = iota16.astype(jnp.float32)

        def scan_body(i, carry):
            a0, b0, lv = carry
            base = i * LANES
            n1 = comb[pl.ds(NBINS + base, LANES)]
            rb = NBINS - base - LANES
            alr = lax.rev(comb[pl.ds(rb, LANES)], (0,))
            fgr = lax.rev(comb[pl.ds(NBINS + rb, LANES)], (0,))
            n0 = alr - fgr
            a_incl = a0 + plsc.cumsum(n1)
            b_incl = b0 + plsc.cumsum(n0)
            a_excl = a_incl - n1
            b_excl = b_incl - n0
            j_incl = 1.0 - (G - a_incl) / jnp.maximum(G + b_incl, 1.0)
            j_excl = 1.0 - (G - a_excl) / jnp.maximum(G + b_excl, 1.0)
            kf = jnp.float32(base) + lanes_f
            center = 1.0 - (kf + 0.5) * jnp.float32(1.0 / NBINS)
            lv = lv + center * (j_incl - j_excl)
            return (a0 + jnp.sum(n1), b0 + jnp.sum(n0), lv)

        _, _, lv = lax.fori_loop(
            0, NBINS // LANES, scan_body,
            (jnp.float32(0.0), jnp.float32(0.0), zeros16))
        loss_c = jnp.sum(lv)
        outv[...] = jnp.where(iota16 == 0, loss_c,
                              jnp.where(iota16 == 1, G, 0.0))
        pltpu.sync_copy(outv, out_hbm.at[pl.ds(cmine * LANES, LANES)])


_sc_call = pl.kernel(
    _body,
    out_type=jax.ShapeDtypeStruct((NCLASS * LANES,), jnp.float32),
    mesh=plsc.VectorSubcoreMesh(core_axis_name="c", subcore_axis_name="s"),
    compiler_params=pltpu.CompilerParams(needs_layout_passes=False),
    scratch_types=[
        pltpu.VMEM((HSZ,), jnp.float32),
        pltpu.VMEM((CS,), jnp.int32),
        pltpu.VMEM((CS,), jnp.float32),
        pltpu.VMEM((CB,), jnp.float32),
        pltpu.VMEM((CB,), jnp.float32),
        pltpu.VMEM((LANES,), jnp.float32),
        pltpu.VMEM_SHARED((NSUB * HSZ,), jnp.float32),
    ],
)


def kernel(y_pred, targets):
    yp = y_pred.reshape(-1)
    lab = targets.reshape(-1)
    out = _sc_call(yp, lab).reshape(NCLASS, LANES)
    losses = out[:, 0]
    present = (out[:, 1] > 0).astype(jnp.float32)
    return jnp.sum(losses * present) / jnp.maximum(jnp.sum(present), 1.0)


# speedup vs baseline: 107.1619x; 2.1608x over previous
import jax
import jax.numpy as jnp
from jax import lax
from jax.experimental import pallas as pl
from jax.experimental.pallas import tpu as pltpu
from jax.experimental.pallas import tpu_sc as plsc

NBINS = 1024
NCORES = 2
NSUB = 16
LANES = 16
NCLASS = 21
CPC = 11
HPP = 512 * 512
NIMG = 8
NPIX = NIMG * HPP
PIX_PER_TILE = NPIX // NSUB
CS = 4096
NCHUNK = PIX_PER_TILE // CS
VPC = CS // LANES
HSZ = CPC * 2 * NBINS
CB = 2 * NBINS


def _body(yp_hbm, lab_hbm, out_hbm, hist, labv, pv, comb, tmp, outv, shared):
    core = lax.axis_index("c")
    sub = lax.axis_index("s")

    zeros16 = jnp.zeros((LANES,), jnp.float32)
    ones16 = jnp.ones((LANES,), jnp.float32)
    iota16 = lax.iota(jnp.int32, LANES)

    def zero_hist(i, _):
        hist[pl.ds(i * LANES, LANES)] = zeros16
        return 0

    lax.fori_loop(0, HSZ // LANES, zero_hist, 0)

    pixbase = sub * PIX_PER_TILE
    img = pixbase // HPP
    poff = pixbase - img * HPP

    def chunk_body(j, _):
        pltpu.sync_copy(lab_hbm.at[pl.ds(pixbase + j * CS, CS)], labv)
        for ci in range(CPC):
            cdyn = core * CPC + ci

            @pl.when(cdyn < NCLASS)
            def _process():
                yp_off = (img * NCLASS + cdyn) * HPP + poff + j * CS
                pltpu.sync_copy(yp_hbm.at[pl.ds(yp_off, CS)], pv)
                base_all = ci * CB

                @plsc.parallel_loop(0, CS, LANES, unroll=8)
                def _vbody(v):
                    p = pv[pl.ds(v, LANES)]
                    labs = labv[pl.ds(v, LANES)]
                    k = jnp.minimum((p * NBINS).astype(jnp.int32), NBINS - 1)
                    idx = k + base_all
                    plsc.addupdate_scatter(hist, [idx], ones16)
                    plsc.addupdate_scatter(hist, [idx + NBINS], ones16,
                                           mask=labs == cdyn)

        return 0

    lax.fori_loop(0, NCHUNK, chunk_body, 0)

    pltpu.sync_copy(hist, shared.at[pl.ds(sub * HSZ, HSZ)])
    plsc.subcore_barrier()

    cmine = core * CPC + sub

    @pl.when(jnp.logical_and(sub < CPC, cmine < NCLASS))
    def _scan():
        def zero_comb(i, _):
            comb[pl.ds(i * LANES, LANES)] = zeros16
            return 0

        lax.fori_loop(0, CB // LANES, zero_comb, 0)

        def tile_sum(t, _):
            pltpu.sync_copy(shared.at[pl.ds(t * HSZ + sub * CB, CB)], tmp)

            def add1(i, _):
                comb[pl.ds(i * LANES, LANES)] += tmp[pl.ds(i * LANES, LANES)]
                return 0

            lax.fori_loop(0, CB // LANES, add1, 0)
            return 0

        lax.fori_loop(0, NSUB, tile_sum, 0)

        def gsum(i, acc):
            return acc + jnp.sum(comb[pl.ds(NBINS + i * LANES, LANES)])

        G = lax.fori_loop(0, NBINS // LANES, gsum, jnp.float32(0.0))

        lanes_f = iota16.astype(jnp.float32)

        def scan_body(i, carry):
            a0, b0, lv = carry
            base = i * LANES
            n1 = comb[pl.ds(NBINS + base, LANES)]
            rb = NBINS - base - LANES
            alr = lax.rev(comb[pl.ds(rb, LANES)], (0,))
            fgr = lax.rev(comb[pl.ds(NBINS + rb, LANES)], (0,))
            n0 = alr - fgr
            a_incl = a0 + plsc.cumsum(n1)
            b_incl = b0 + plsc.cumsum(n0)
            a_excl = a_incl - n1
            b_excl = b_incl - n0
            j_incl = 1.0 - (G - a_incl) / jnp.maximum(G + b_incl, 1.0)
            j_excl = 1.0 - (G - a_excl) / jnp.maximum(G + b_excl, 1.0)
            kf = jnp.float32(base) + lanes_f
            center = 1.0 - (kf + 0.5) * jnp.float32(1.0 / NBINS)
            lv = lv + center * (j_incl - j_excl)
            return (a0 + jnp.sum(n1), b0 + jnp.sum(n0), lv)

        _, _, lv = lax.fori_loop(
            0, NBINS // LANES, scan_body,
            (jnp.float32(0.0), jnp.float32(0.0), zeros16))
        loss_c = jnp.sum(lv)
        outv[...] = jnp.where(iota16 == 0, loss_c,
                              jnp.where(iota16 == 1, G, 0.0))
        pltpu.sync_copy(outv, out_hbm.at[pl.ds(cmine * LANES, LANES)])


_sc_call = pl.kernel(
    _body,
    out_type=jax.ShapeDtypeStruct((NCLASS * LANES,), jnp.float32),
    mesh=plsc.VectorSubcoreMesh(core_axis_name="c", subcore_axis_name="s"),
    compiler_params=pltpu.CompilerParams(needs_layout_passes=False),
    scratch_types=[
        pltpu.VMEM((HSZ,), jnp.float32),
        pltpu.VMEM((CS,), jnp.int32),
        pltpu.VMEM((CS,), jnp.float32),
        pltpu.VMEM((CB,), jnp.float32),
        pltpu.VMEM((CB,), jnp.float32),
        pltpu.VMEM((LANES,), jnp.float32),
        pltpu.VMEM_SHARED((NSUB * HSZ,), jnp.float32),
    ],
)


def kernel(y_pred, targets):
    yp = y_pred.reshape(-1)
    lab = targets.reshape(-1)
    out = _sc_call(yp, lab).reshape(NCLASS, LANES)
    losses = out[:, 0]
    present = (out[:, 1] > 0).astype(jnp.float32)
    return jnp.sum(losses * present) / jnp.maximum(jnp.sum(present), 1.0)


# speedup vs baseline: 127.1004x; 1.1861x over previous
import jax
import jax.numpy as jnp
from jax import lax
from jax.experimental import pallas as pl
from jax.experimental.pallas import tpu as pltpu
from jax.experimental.pallas import tpu_sc as plsc

NBINS = 1024
NCORES = 2
NSUB = 16
LANES = 16
NCLASS = 21
CPC = 11
HPP = 512 * 512
NIMG = 8
NPIX = NIMG * HPP
PIX_PER_TILE = NPIX // NSUB
CS = 8192
NCHUNK = PIX_PER_TILE // CS
VPC = CS // LANES
HSZ = CPC * 2 * NBINS
CB = 2 * NBINS


def _body(yp_hbm, lab_hbm, out_hbm, hist, labv, pv, comb, tmp, outv, shared):
    core = lax.axis_index("c")
    sub = lax.axis_index("s")

    zeros16 = jnp.zeros((LANES,), jnp.float32)
    ones16 = jnp.ones((LANES,), jnp.float32)
    iota16 = lax.iota(jnp.int32, LANES)

    def zero_hist(i, _):
        hist[pl.ds(i * LANES, LANES)] = zeros16
        return 0

    lax.fori_loop(0, HSZ // LANES, zero_hist, 0)

    pixbase = sub * PIX_PER_TILE
    img = pixbase // HPP
    poff = pixbase - img * HPP

    def chunk_body(j, _):
        pltpu.sync_copy(lab_hbm.at[pl.ds(pixbase + j * CS, CS)], labv)
        for ci in range(CPC):
            cdyn = core * CPC + ci

            @pl.when(cdyn < NCLASS)
            def _process():
                yp_off = (img * NCLASS + cdyn) * HPP + poff + j * CS
                pltpu.sync_copy(yp_hbm.at[pl.ds(yp_off, CS)], pv)
                base_all = ci * CB

                @plsc.parallel_loop(0, CS, LANES, unroll=16)
                def _vbody(v):
                    p = pv[pl.ds(v, LANES)]
                    labs = labv[pl.ds(v, LANES)]
                    k = jnp.minimum((p * NBINS).astype(jnp.int32), NBINS - 1)
                    idx = k + base_all
                    plsc.addupdate_scatter(hist, [idx], ones16)
                    plsc.addupdate_scatter(hist, [idx + NBINS], ones16,
                                           mask=labs == cdyn)

        return 0

    lax.fori_loop(0, NCHUNK, chunk_body, 0)

    pltpu.sync_copy(hist, shared.at[pl.ds(sub * HSZ, HSZ)])
    plsc.subcore_barrier()

    cmine = core * CPC + sub

    @pl.when(jnp.logical_and(sub < CPC, cmine < NCLASS))
    def _scan():
        def zero_comb(i, _):
            comb[pl.ds(i * LANES, LANES)] = zeros16
            return 0

        lax.fori_loop(0, CB // LANES, zero_comb, 0)

        def tile_sum(t, _):
            pltpu.sync_copy(shared.at[pl.ds(t * HSZ + sub * CB, CB)], tmp)

            def add1(i, _):
                comb[pl.ds(i * LANES, LANES)] += tmp[pl.ds(i * LANES, LANES)]
                return 0

            lax.fori_loop(0, CB // LANES, add1, 0)
            return 0

        lax.fori_loop(0, NSUB, tile_sum, 0)

        def gsum(i, acc):
            return acc + jnp.sum(comb[pl.ds(NBINS + i * LANES, LANES)])

        G = lax.fori_loop(0, NBINS // LANES, gsum, jnp.float32(0.0))

        lanes_f = iota16.astype(jnp.float32)

        def scan_body(i, carry):
            a0, b0, lv = carry
            base = i * LANES
            n1 = comb[pl.ds(NBINS + base, LANES)]
            rb = NBINS - base - LANES
            alr = lax.rev(comb[pl.ds(rb, LANES)], (0,))
            fgr = lax.rev(comb[pl.ds(NBINS + rb, LANES)], (0,))
            n0 = alr - fgr
            a_incl = a0 + plsc.cumsum(n1)
            b_incl = b0 + plsc.cumsum(n0)
            a_excl = a_incl - n1
            b_excl = b_incl - n0
            j_incl = 1.0 - (G - a_incl) / jnp.maximum(G + b_incl, 1.0)
            j_excl = 1.0 - (G - a_excl) / jnp.maximum(G + b_excl, 1.0)
            kf = jnp.float32(base) + lanes_f
            center = 1.0 - (kf + 0.5) * jnp.float32(1.0 / NBINS)
            lv = lv + center * (j_incl - j_excl)
            return (a0 + jnp.sum(n1), b0 + jnp.sum(n0), lv)

        _, _, lv = lax.fori_loop(
            0, NBINS // LANES, scan_body,
            (jnp.float32(0.0), jnp.float32(0.0), zeros16))
        loss_c = jnp.sum(lv)
        outv[...] = jnp.where(iota16 == 0, loss_c,
                              jnp.where(iota16 == 1, G, 0.0))
        pltpu.sync_copy(outv, out_hbm.at[pl.ds(cmine * LANES, LANES)])


_sc_call = pl.kernel(
    _body,
    out_type=jax.ShapeDtypeStruct((NCLASS * LANES,), jnp.float32),
    mesh=plsc.VectorSubcoreMesh(core_axis_name="c", subcore_axis_name="s"),
    compiler_params=pltpu.CompilerParams(needs_layout_passes=False),
    scratch_types=[
        pltpu.VMEM((HSZ,), jnp.float32),
        pltpu.VMEM((CS,), jnp.int32),
        pltpu.VMEM((CS,), jnp.float32),
        pltpu.VMEM((CB,), jnp.float32),
        pltpu.VMEM((CB,), jnp.float32),
        pltpu.VMEM((LANES,), jnp.float32),
        pltpu.VMEM_SHARED((NSUB * HSZ,), jnp.float32),
    ],
)


def kernel(y_pred, targets):
    yp = y_pred.reshape(-1)
    lab = targets.reshape(-1)
    out = _sc_call(yp, lab).reshape(NCLASS, LANES)
    losses = out[:, 0]
    present = (out[:, 1] > 0).astype(jnp.float32)
    return jnp.sum(losses * present) / jnp.maximum(jnp.sum(present), 1.0)


# speedup vs baseline: 175.8659x; 1.3837x over previous
import jax
import jax.numpy as jnp
from jax import lax
from jax.experimental import pallas as pl
from jax.experimental.pallas import tpu as pltpu
from jax.experimental.pallas import tpu_sc as plsc

NBINS = 1024
NCORES = 2
NSUB = 16
LANES = 16
NCLASS = 21
CPC = 11
HPP = 512 * 512
NIMG = 8
NPIX = NIMG * HPP
PIX_PER_TILE = NPIX // NSUB
CS = 16384
NCHUNK = PIX_PER_TILE // CS
HSZ = CPC * 2 * NBINS
CB = 2 * NBINS


def _body(yp_hbm, lab_hbm, out_hbm, hist, labA, labB, pvA, pvB, comb, tmp,
          outv, shared, sem_l, sem_p):
    core = lax.axis_index("c")
    sub = lax.axis_index("s")

    zeros16 = jnp.zeros((LANES,), jnp.float32)
    ones16 = jnp.ones((LANES,), jnp.float32)
    iota16 = lax.iota(jnp.int32, LANES)

    def zero_hist(i, _):
        hist[pl.ds(i * LANES, LANES)] = zeros16
        return 0

    lax.fori_loop(0, HSZ // LANES, zero_hist, 0)

    pixbase = sub * PIX_PER_TILE
    img = pixbase // HPP
    poff = pixbase - img * HPP

    def yp_off(j, c):
        return (img * NCLASS + c) * HPP + poff + j * CS

    pltpu.async_copy(lab_hbm.at[pl.ds(pixbase, CS)], labA, sem_l)
    c0 = core * CPC
    pltpu.async_copy(yp_hbm.at[pl.ds(yp_off(0, c0), CS)], pvA, sem_p)

    def chunk_pair(jj, _):
        j0 = jj * 2
        for dj in range(2):
            j = j0 + dj
            labcur, labnxt = (labA, labB) if dj == 0 else (labB, labA)
            pltpu.make_async_copy(
                lab_hbm.at[pl.ds(0, CS)], labcur, sem_l).wait()

            @pl.when(j + 1 < NCHUNK)
            def _issue_lab():
                pltpu.async_copy(
                    lab_hbm.at[pl.ds(pixbase + (j + 1) * CS, CS)],
                    labnxt, sem_l)

            for ci in range(CPC):
                cdyn = core * CPC + ci
                par = (dj * CPC + ci) % 2
                cur, nxt = (pvA, pvB) if par == 0 else (pvB, pvA)

                pltpu.make_async_copy(
                    yp_hbm.at[pl.ds(0, CS)], cur, sem_p).wait()

                if ci < CPC - 1:
                    cnxt = jnp.minimum(cdyn + 1, NCLASS - 1)
                    pltpu.async_copy(
                        yp_hbm.at[pl.ds(yp_off(j, cnxt), CS)], nxt, sem_p)
                else:
                    @pl.when(j + 1 < NCHUNK)
                    def _issue_yp():
                        pltpu.async_copy(
                            yp_hbm.at[pl.ds(yp_off(j + 1, c0), CS)],
                            nxt, sem_p)

                @pl.when(cdyn < NCLASS)
                def _process():
                    base_all = ci * CB

                    @plsc.parallel_loop(0, CS, LANES, unroll=16)
                    def _vbody(v):
                        p = cur[pl.ds(v, LANES)]
                        labs = labcur[pl.ds(v, LANES)]
                        k = jnp.minimum((p * NBINS).astype(jnp.int32),
                                        NBINS - 1)
                        idx = k + base_all
                        plsc.addupdate_scatter(hist, [idx], ones16)
                        plsc.addupdate_scatter(hist, [idx + NBINS], ones16,
                                               mask=labs == cdyn)

        return 0

    lax.fori_loop(0, NCHUNK // 2, chunk_pair, 0)

    pltpu.sync_copy(hist, shared.at[pl.ds(sub * HSZ, HSZ)])
    plsc.subcore_barrier()

    cmine = core * CPC + sub

    @pl.when(jnp.logical_and(sub < CPC, cmine < NCLASS))
    def _scan():
        def zero_comb(i, _):
            comb[pl.ds(i * LANES, LANES)] = zeros16
            return 0

        lax.fori_loop(0, CB // LANES, zero_comb, 0)

        def tile_sum(t, _):
            pltpu.sync_copy(shared.at[pl.ds(t * HSZ + sub * CB, CB)], tmp)

            def add1(i, _):
                comb[pl.ds(i * LANES, LANES)] += tmp[pl.ds(i * LANES, LANES)]
                return 0

            lax.fori_loop(0, CB // LANES, add1, 0)
            return 0

        lax.fori_loop(0, NSUB, tile_sum, 0)

        def gsum(i, acc):
            return acc + jnp.sum(comb[pl.ds(NBINS + i * LANES, LANES)])

        G = lax.fori_loop(0, NBINS // LANES, gsum, jnp.float32(0.0))

        lanes_f = iota16.astype(jnp.float32)

        def scan_body(i, carry):
            a0, b0, lv = carry
            base = i * LANES
            n1 = comb[pl.ds(NBINS + base, LANES)]
            rb = NBINS - base - LANES
            alr = lax.rev(comb[pl.ds(rb, LANES)], (0,))
            fgr = lax.rev(comb[pl.ds(NBINS + rb, LANES)], (0,))
            n0 = alr - fgr
            a_incl = a0 + plsc.cumsum(n1)
            b_incl = b0 + plsc.cumsum(n0)
            a_excl = a_incl - n1
            b_excl = b_incl - n0
            j_incl = 1.0 - (G - a_incl) / jnp.maximum(G + b_incl, 1.0)
            j_excl = 1.0 - (G - a_excl) / jnp.maximum(G + b_excl, 1.0)
            kf = jnp.float32(base) + lanes_f
            center = 1.0 - (kf + 0.5) * jnp.float32(1.0 / NBINS)
            lv = lv + center * (j_incl - j_excl)
            return (a0 + jnp.sum(n1), b0 + jnp.sum(n0), lv)

        _, _, lv = lax.fori_loop(
            0, NBINS // LANES, scan_body,
            (jnp.float32(0.0), jnp.float32(0.0), zeros16))
        loss_c = jnp.sum(lv)
        outv[...] = jnp.where(iota16 == 0, loss_c,
                              jnp.where(iota16 == 1, G, 0.0))
        pltpu.sync_copy(outv, out_hbm.at[pl.ds(cmine * LANES, LANES)])


_sc_call = pl.kernel(
    _body,
    out_type=jax.ShapeDtypeStruct((NCLASS * LANES,), jnp.float32),
    mesh=plsc.VectorSubcoreMesh(core_axis_name="c", subcore_axis_name="s"),
    compiler_params=pltpu.CompilerParams(needs_layout_passes=False),
    scratch_types=[
        pltpu.VMEM((HSZ,), jnp.float32),
        pltpu.VMEM((CS,), jnp.int32),
        pltpu.VMEM((CS,), jnp.int32),
        pltpu.VMEM((CS,), jnp.float32),
        pltpu.VMEM((CS,), jnp.float32),
        pltpu.VMEM((CB,), jnp.float32),
        pltpu.VMEM((CB,), jnp.float32),
        pltpu.VMEM((LANES,), jnp.float32),
        pltpu.VMEM_SHARED((NSUB * HSZ,), jnp.float32),
        pltpu.SemaphoreType.DMA,
        pltpu.SemaphoreType.DMA,
    ],
)


def kernel(y_pred, targets):
    yp = y_pred.reshape(-1)
    lab = targets.reshape(-1)
    out = _sc_call(yp, lab).reshape(NCLASS, LANES)
    losses = out[:, 0]
    present = (out[:, 1] > 0).astype(jnp.float32)
    return jnp.sum(losses * present) / jnp.maximum(jnp.sum(present), 1.0)


# speedup vs baseline: 187.9051x; 1.0685x over previous
import jax
import jax.numpy as jnp
from jax import lax
from jax.experimental import pallas as pl
from jax.experimental.pallas import tpu as pltpu
from jax.experimental.pallas import tpu_sc as plsc

NBINS = 1024
NCORES = 2
NSUB = 16
LANES = 16
NCLASS = 21
CPC = 11
HPP = 512 * 512
NIMG = 8
NPIX = NIMG * HPP
PIX_PER_TILE = NPIX // NSUB
CS = 16384
NCHUNK = PIX_PER_TILE // CS
HSZ = CPC * 2 * NBINS
CB = 2 * NBINS


def _body(yp_hbm, lab_hbm, out_hbm, hist, labA, labB, pvA, pvB, comb, tmp,
          outv, shared, sem_l, sem_p):
    core = lax.axis_index("c")
    sub = lax.axis_index("s")

    zeros16 = jnp.zeros((LANES,), jnp.float32)
    ones16 = jnp.ones((LANES,), jnp.float32)
    iota16 = lax.iota(jnp.int32, LANES)

    def zero_hist(i, _):
        hist[pl.ds(i * LANES, LANES)] = zeros16
        return 0

    lax.fori_loop(0, HSZ // LANES, zero_hist, 0)

    pixbase = sub * PIX_PER_TILE
    img = pixbase // HPP
    poff = pixbase - img * HPP

    def yp_off(j, c):
        return (img * NCLASS + c) * HPP + poff + j * CS

    pltpu.async_copy(lab_hbm.at[pl.ds(pixbase, CS)], labA, sem_l)
    c0 = core * CPC
    pltpu.async_copy(yp_hbm.at[pl.ds(yp_off(0, c0), CS)], pvA, sem_p)

    def chunk_pair(jj, _):
        j0 = jj * 2
        for dj in range(2):
            j = j0 + dj
            labcur, labnxt = (labA, labB) if dj == 0 else (labB, labA)
            pltpu.make_async_copy(
                lab_hbm.at[pl.ds(0, CS)], labcur, sem_l).wait()

            @pl.when(j + 1 < NCHUNK)
            def _issue_lab():
                pltpu.async_copy(
                    lab_hbm.at[pl.ds(pixbase + (j + 1) * CS, CS)],
                    labnxt, sem_l)

            for ci in range(CPC):
                cdyn = core * CPC + ci
                par = (dj * CPC + ci) % 2
                cur, nxt = (pvA, pvB) if par == 0 else (pvB, pvA)

                pltpu.make_async_copy(
                    yp_hbm.at[pl.ds(0, CS)], cur, sem_p).wait()

                if ci < CPC - 1:
                    cnxt = jnp.minimum(cdyn + 1, NCLASS - 1)
                    pltpu.async_copy(
                        yp_hbm.at[pl.ds(yp_off(j, cnxt), CS)], nxt, sem_p)
                else:
                    @pl.when(j + 1 < NCHUNK)
                    def _issue_yp():
                        pltpu.async_copy(
                            yp_hbm.at[pl.ds(yp_off(j + 1, c0), CS)],
                            nxt, sem_p)

                @pl.when(cdyn < NCLASS)
                def _process():
                    base_bg = ci * CB
                    base_fg = base_bg + NBINS
                    scale = jnp.float32(NBINS - 1.0 / 64.0)

                    @plsc.parallel_loop(0, CS, LANES, unroll=16)
                    def _vbody(v):
                        p = cur[pl.ds(v, LANES)]
                        labs = labcur[pl.ds(v, LANES)]
                        k = (p * scale).astype(jnp.int32)
                        base = jnp.where(labs == cdyn, base_fg, base_bg)
                        plsc.addupdate_scatter(hist, [k + base], ones16)

        return 0

    lax.fori_loop(0, NCHUNK // 2, chunk_pair, 0)

    pltpu.sync_copy(hist, shared.at[pl.ds(sub * HSZ, HSZ)])
    plsc.subcore_barrier()

    cmine = core * CPC + sub

    @pl.when(jnp.logical_and(sub < CPC, cmine < NCLASS))
    def _scan():
        def zero_comb(i, _):
            comb[pl.ds(i * LANES, LANES)] = zeros16
            return 0

        lax.fori_loop(0, CB // LANES, zero_comb, 0)

        def tile_sum(t, _):
            pltpu.sync_copy(shared.at[pl.ds(t * HSZ + sub * CB, CB)], tmp)

            def add1(i, _):
                comb[pl.ds(i * LANES, LANES)] += tmp[pl.ds(i * LANES, LANES)]
                return 0

            lax.fori_loop(0, CB // LANES, add1, 0)
            return 0

        lax.fori_loop(0, NSUB, tile_sum, 0)

        def gsum(i, acc):
            return acc + jnp.sum(comb[pl.ds(NBINS + i * LANES, LANES)])

        G = lax.fori_loop(0, NBINS // LANES, gsum, jnp.float32(0.0))

        lanes_f = iota16.astype(jnp.float32)

        def scan_body(i, carry):
            a0, b0, lv = carry
            base = i * LANES
            n1 = comb[pl.ds(NBINS + base, LANES)]
            rb = NBINS - base - LANES
            n0 = lax.rev(comb[pl.ds(rb, LANES)], (0,))
            a_incl = a0 + plsc.cumsum(n1)
            b_incl = b0 + plsc.cumsum(n0)
            a_excl = a_incl - n1
            b_excl = b_incl - n0
            j_incl = 1.0 - (G - a_incl) / jnp.maximum(G + b_incl, 1.0)
            j_excl = 1.0 - (G - a_excl) / jnp.maximum(G + b_excl, 1.0)
            kf = jnp.float32(base) + lanes_f
            center = 1.0 - (kf + 0.5) * jnp.float32(1.0 / NBINS)
            lv = lv + center * (j_incl - j_excl)
            return (a0 + jnp.sum(n1), b0 + jnp.sum(n0), lv)

        _, _, lv = lax.fori_loop(
            0, NBINS // LANES, scan_body,
            (jnp.float32(0.0), jnp.float32(0.0), zeros16))
        loss_c = jnp.sum(lv)
        outv[...] = jnp.where(iota16 == 0, loss_c,
                              jnp.where(iota16 == 1, G, 0.0))
        pltpu.sync_copy(outv, out_hbm.at[pl.ds(cmine * LANES, LANES)])


_sc_call = pl.kernel(
    _body,
    out_type=jax.ShapeDtypeStruct((NCLASS * LANES,), jnp.float32),
    mesh=plsc.VectorSubcoreMesh(core_axis_name="c", subcore_axis_name="s"),
    compiler_params=pltpu.CompilerParams(needs_layout_passes=False),
    scratch_types=[
        pltpu.VMEM((HSZ,), jnp.float32),
        pltpu.VMEM((CS,), jnp.int32),
        pltpu.VMEM((CS,), jnp.int32),
        pltpu.VMEM((CS,), jnp.float32),
        pltpu.VMEM((CS,), jnp.float32),
        pltpu.VMEM((CB,), jnp.float32),
        pltpu.VMEM((CB,), jnp.float32),
        pltpu.VMEM((LANES,), jnp.float32),
        pltpu.VMEM_SHARED((NSUB * HSZ,), jnp.float32),
        pltpu.SemaphoreType.DMA,
        pltpu.SemaphoreType.DMA,
    ],
)


def kernel(y_pred, targets):
    yp = y_pred.reshape(-1)
    lab = targets.reshape(-1)
    out = _sc_call(yp, lab).reshape(NCLASS, LANES)
    losses = out[:, 0]
    present = (out[:, 1] > 0).astype(jnp.float32)
    return jnp.sum(losses * present) / jnp.maximum(jnp.sum(present), 1.0)
